# Initial kernel scaffold; baseline (speedup 1.0000x reference)
import jax
import jax.numpy as jnp
from jax.experimental import pallas as pl


def kernel(x, edge_index, edge_attr, params):
    raise NotImplementedError("write your pallas kernel here")


# speedup vs baseline: 5.7799x; 5.7799x over previous
import functools

import jax
import jax.numpy as jnp
from jax import lax
from jax.experimental import pallas as pl
from jax.experimental.pallas import tpu as pltpu
from jax.experimental.pallas import tpu_sc as plsc

N = 10000
T = 2
IN_F = 128
DH = 256
ED = 16
E = 160000
NT = N * T
LCFG = [(4, 64, True), (1, 256, False)]

CH = N // 2
ACC_ROWS = 5120
ACC_W = 272
LS_W = 128
EB = 80
EBE = 16
BPT = (E // EB) // 16
BPTE = (E // EBE) // 16
ROWS_PT = ACC_ROWS // 16
RB = 1000

_mesh = functools.partial(
    plsc.VectorSubcoreMesh, core_axis_name="c", subcore_axis_name="s")


def _elu(x):
    return jnp.where(x > 0, x, jnp.exp(x) - 1.0)


def _ln(x, g, b):
    m = jnp.mean(x, axis=-1, keepdims=True)
    v = jnp.mean((x - m) ** 2, axis=-1, keepdims=True)
    return (x - m) / jnp.sqrt(v + 1e-5) * g + b


def _k1_body(x_ref, w_ref, b_ref, g_ref, be_ref, o_ref):
    h = jnp.dot(x_ref[...], w_ref[...], preferred_element_type=jnp.float32)
    h = h + b_ref[...]
    o_ref[...] = _elu(_ln(h, g_ref[...], be_ref[...]))


def _k1(x2, w, b, g, be):
    return pl.pallas_call(
        _k1_body,
        grid=(NT // RB,),
        in_specs=[
            pl.BlockSpec((RB, IN_F), lambda i: (i, 0)),
            pl.BlockSpec((IN_F, DH), lambda i: (0, 0)),
            pl.BlockSpec((1, DH), lambda i: (0, 0)),
            pl.BlockSpec((1, DH), lambda i: (0, 0)),
            pl.BlockSpec((1, DH), lambda i: (0, 0)),
        ],
        out_specs=pl.BlockSpec((RB, DH), lambda i: (i, 0)),
        out_shape=jax.ShapeDtypeStruct((NT, DH), jnp.float32),
    )(x2, w, b, g, be)


def _k2_body(h_ref, wl_ref, bl_ref, wr_ref, br_ref, ol_ref, or_ref):
    h = h_ref[...]
    ol_ref[...] = jnp.dot(h, wl_ref[...],
                          preferred_element_type=jnp.float32) + bl_ref[...]
    or_ref[...] = jnp.dot(h, wr_ref[...],
                          preferred_element_type=jnp.float32) + br_ref[...]


def _k2(h, wl, bl, wr, br):
    return pl.pallas_call(
        _k2_body,
        grid=(NT // RB,),
        in_specs=[
            pl.BlockSpec((RB, DH), lambda i: (i, 0)),
            pl.BlockSpec((DH, DH), lambda i: (0, 0)),
            pl.BlockSpec((1, DH), lambda i: (0, 0)),
            pl.BlockSpec((DH, DH), lambda i: (0, 0)),
            pl.BlockSpec((1, DH), lambda i: (0, 0)),
        ],
        out_specs=[
            pl.BlockSpec((RB, DH), lambda i: (i, 0)),
            pl.BlockSpec((RB, DH), lambda i: (i, 0)),
        ],
        out_shape=[
            jax.ShapeDtypeStruct((NT, DH), jnp.float32),
            jax.ShapeDtypeStruct((NT, DH), jnp.float32),
        ],
    )(h, wl, bl, wr, br)


_EB_TC = 2000


def _k3_body(ea_ref, we_ref, o_ref):
    o_ref[...] = jnp.dot(ea_ref[...], we_ref[...],
                         preferred_element_type=jnp.float32)


def _k3(ea, we):
    return pl.pallas_call(
        _k3_body,
        grid=(E // _EB_TC,),
        in_specs=[
            pl.BlockSpec((_EB_TC, ED), lambda i: (i, 0)),
            pl.BlockSpec((ED, DH), lambda i: (0, 0)),
        ],
        out_specs=pl.BlockSpec((_EB_TC, DH), lambda i: (i, 0)),
        out_shape=jax.ShapeDtypeStruct((E, DH), jnp.float32),
    )(ea, we)


N_LS = 10240
_LS_ROWS_PT = N_LS // 16


def _loopstats_body(dst_hbm, ea_hbm, out_hbm, dstb, eab, pay, stat_sh, sem):
    c = lax.axis_index("c")
    s = lax.axis_index("s")
    zero16 = jnp.zeros((16,), jnp.float32)
    one16 = jnp.where(lax.broadcasted_iota(jnp.int32, (16,), 0) == 0,
                      1.0, 0.0).astype(jnp.float32)

    @pl.when(c == 0)
    def _():
        def zfill(i, _):
            for k in range(LS_W // 16):
                pay[i, pl.ds(k * 16, 16)] = zero16
            return 0
        lax.fori_loop(0, EB, zfill, 0)

        def zcopy(q, _):
            pltpu.sync_copy(pay,
                            stat_sh.at[pl.ds(s * _LS_ROWS_PT + q * EB, EB)])
            return 0
        lax.fori_loop(0, _LS_ROWS_PT // EB, zcopy, 0)

        def fill(i, _):
            pay[i, pl.ds(32, 16)] = one16
            return 0
        lax.fori_loop(0, EB, fill, 0)
        plsc.subcore_barrier()

        def blk(i, _):
            base = s * (E // 16) + i * EB
            off2 = 2 * base - jnp.where(s >= 8, E, 0)
            pltpu.sync_copy(dst_hbm.at[pl.ds(base, EB)], dstb)
            pltpu.sync_copy(ea_hbm.at[pl.ds(off2, 2 * EB)], eab)

            def edge(j, _):
                pay[j, pl.ds(0, 16)] = eab[2 * j, pl.ds(0, 16)]
                pay[j, pl.ds(16, 16)] = eab[2 * j + 1, pl.ds(0, 16)]
                return 0
            lax.fori_loop(0, EB, edge, 0)
            pltpu.sync_copy(pay, stat_sh.at[dstb], add=True)
            return 0
        lax.fori_loop(0, BPT, blk, 0)
        plsc.subcore_barrier()
        pltpu.sync_copy(stat_sh.at[pl.ds(s * _LS_ROWS_PT, _LS_ROWS_PT)],
                        out_hbm.at[pl.ds(s * _LS_ROWS_PT, _LS_ROWS_PT)])


def _loopstats(dst, ea):
    return pl.kernel(
        _loopstats_body,
        out_type=jax.ShapeDtypeStruct((N_LS, LS_W), jnp.float32),
        mesh=_mesh(),
        scratch_types=[
            pltpu.VMEM((EB,), jnp.int32),
            pltpu.VMEM((2 * EB, 16), jnp.float32),
            pltpu.VMEM((EB, LS_W), jnp.float32),
            pltpu.VMEM_SHARED((N_LS, LS_W), jnp.float32),
            pltpu.SemaphoreType.DMA,
        ],
    )(dst, ea)


_GTR_DNUMS = lax.GatherDimensionNumbers(
    offset_dims=(), collapsed_slice_dims=(0,), start_index_map=(0,))


def _lane_perm(x, idx):
    return lax.gather(x, idx[:, None], _GTR_DNUMS, (1,),
                      mode=lax.GatherScatterMode.PROMISE_IN_BOUNDS)


def _lane_sum(x):
    iota = lax.broadcasted_iota(jnp.int32, (16,), 0)
    for sh in (8, 4, 2, 1):
        x = x + _lane_perm(x, iota ^ sh)
    return x


def _edge_body(CPH, xl_hbm, xr_hbm, ep_hbm, att_hbm, src_hbm, dst_hbm,
               out_hbm, srcb, dstb, jidx, iidx, eidx, didx, xjb, xib, eb, vb,
               attv, acc_sh, sem):
    c = lax.axis_index("c")
    s = lax.axis_index("s")
    H = 16 // CPH
    lo = c * CH
    zero16 = jnp.zeros((16,), jnp.float32)
    iota16 = lax.broadcasted_iota(jnp.int32, (16,), 0)
    pltpu.sync_copy(att_hbm, attv)

    for t in range(T):
        toff = t * N

        def zrow(i, _):
            for k in range(ACC_W // 16):
                vb[i, pl.ds(k * 16, 16)] = zero16
            return 0
        lax.fori_loop(0, EBE, zrow, 0)
        for q in range(ROWS_PT // EBE):
            pltpu.sync_copy(vb, acc_sh.at[pl.ds(s * ROWS_PT + q * EBE, EBE)])
        plsc.subcore_barrier()

        def blk(i, _):
            base = s * (E // 16) + i * EBE
            pltpu.sync_copy(src_hbm.at[pl.ds(base, EBE)], srcb)
            pltpu.sync_copy(dst_hbm.at[pl.ds(base, EBE)], dstb)
            for q in range(EBE // 16):
                sl = pl.ds(q * 16, 16)
                sv = srcb[sl]
                dv = dstb[sl]
                jidx[sl] = sv + toff
                iidx[sl] = dv + toff
                inb = (dv >= lo) & (dv < lo + CH)
                didx[sl] = jnp.where(inb, dv - lo, CH)
                ev = 2 * (base + q * 16 + iota16) + t
                eidx[sl] = jnp.where(ev >= E, ev - E, ev)
            cj = pltpu.async_copy(xl_hbm.at[jidx], xjb, sem)
            ci = pltpu.async_copy(xr_hbm.at[iidx], xib, sem)
            ce = pltpu.async_copy(ep_hbm.at[eidx], eb, sem)
            cj.wait()
            ci.wait()
            ce.wait()

            def edge(j, _):
                ph = [zero16 for _ in range(H)]
                xjs = []
                for k in range(16):
                    sl = pl.ds(k * 16, 16)
                    xj = xjb[j, sl]
                    z = xj + xib[j, sl] + eb[j, sl]
                    m = jnp.maximum(z, 0.2 * z)
                    ph[k // CPH] = ph[k // CPH] + m * attv[sl]
                    xjs.append(xj)
                den = zero16
                wv = []
                for h in range(H):
                    w = jnp.exp(_lane_sum(ph[h]))
                    wv.append(w)
                    den = jnp.where(iota16 == h, w, den)
                for k in range(16):
                    vb[j, pl.ds(k * 16, 16)] = xjs[k] * wv[k // CPH]
                vb[j, pl.ds(256, 16)] = den
                return 0
            lax.fori_loop(0, EBE, edge, 0)
            pltpu.sync_copy(vb, acc_sh.at[didx], add=True)
            return 0
        lax.fori_loop(0, BPTE, blk, 0)
        plsc.subcore_barrier()
        pltpu.sync_copy(acc_sh.at[pl.ds(s * ROWS_PT, ROWS_PT)],
                        out_hbm.at[t, c, pl.ds(s * ROWS_PT, ROWS_PT)])
        plsc.subcore_barrier()


def _edge_kernel(CPH, xl, xr, ep, att, src, dst):
    return pl.kernel(
        functools.partial(_edge_body, CPH),
        out_type=jax.ShapeDtypeStruct((T, 2, ACC_ROWS, ACC_W), jnp.float32),
        mesh=_mesh(),
        scratch_types=[
            pltpu.VMEM((EBE,), jnp.int32),
            pltpu.VMEM((EBE,), jnp.int32),
            pltpu.VMEM((EBE,), jnp.int32),
            pltpu.VMEM((EBE,), jnp.int32),
            pltpu.VMEM((EBE,), jnp.int32),
            pltpu.VMEM((EBE,), jnp.int32),
            pltpu.VMEM((EBE, DH), jnp.float32),
            pltpu.VMEM((EBE, DH), jnp.float32),
            pltpu.VMEM((EBE, DH), jnp.float32),
            pltpu.VMEM((EBE, ACC_W), jnp.float32),
            pltpu.VMEM((DH,), jnp.float32),
            pltpu.VMEM_SHARED((ACC_ROWS, ACC_W), jnp.float32),
            pltpu.SemaphoreType.DMA,
        ],
        compiler_params=pltpu.CompilerParams(use_tc_tiling_on_sc=False),
    )(xl, xr, ep, att, src, dst)


def _k4_body(H, h_ref, nd_ref, xl_ref, xr_ref, st_ref, we_ref, att_ref,
             bias_ref, gn_ref, bn_ref, w1_ref, b1_ref, w2_ref, b2_ref,
             gf_ref, bf_ref, o_ref):
    C = DH // H
    h = h_ref[...]
    nd = nd_ref[...]
    xl = xl_ref[...]
    xr = xr_ref[...]
    st = st_ref[...]
    t_is_1 = pl.program_id(0) >= (N // RB)
    ea_sum = jnp.where(t_is_1, st[:, ED:2 * ED], st[:, :ED])
    ea_m = ea_sum / jnp.clip(st[:, 2 * ED:2 * ED + 1], 1.0)
    el = jnp.dot(ea_m, we_ref[...], preferred_element_type=jnp.float32)
    z = xl + xr + el
    m = jnp.maximum(z, 0.2 * z)
    p = m * att_ref[...]
    pieces = []
    for hh in range(H):
        csl = slice(hh * C, (hh + 1) * C)
        a = jnp.sum(p[:, csl], axis=-1, keepdims=True)
        w = jnp.exp(a)
        numt = nd[:, csl] + w * xl[:, csl]
        dent = nd[:, DH + hh:DH + hh + 1] + w
        pieces.append(numt / dent)
    o = jnp.concatenate(pieces, axis=-1) if H > 1 else pieces[0]
    o = o + bias_ref[...]
    o = _elu(_ln(o, gn_ref[...], bn_ref[...]))
    f = _elu(jnp.dot(o, w1_ref[...],
                     preferred_element_type=jnp.float32) + b1_ref[...])
    f2 = jnp.dot(f, w2_ref[...],
                 preferred_element_type=jnp.float32) + b2_ref[...]
    o_ref[...] = h + _ln(f2, gf_ref[...], bf_ref[...])


def _k4(H, h, nd, xl, xr, st, we, att, bias, gn, bn, w1, b1, w2, b2, gf, bf):
    return pl.pallas_call(
        functools.partial(_k4_body, H),
        grid=(NT // RB,),
        in_specs=[
            pl.BlockSpec((RB, DH), lambda i: (i, 0)),
            pl.BlockSpec((RB, ACC_W), lambda i: (i, 0)),
            pl.BlockSpec((RB, DH), lambda i: (i, 0)),
            pl.BlockSpec((RB, DH), lambda i: (i, 0)),
            pl.BlockSpec((RB, LS_W), lambda i: (i % (N // RB), 0)),
            pl.BlockSpec((ED, DH), lambda i: (0, 0)),
            pl.BlockSpec((1, DH), lambda i: (0, 0)),
            pl.BlockSpec((1, DH), lambda i: (0, 0)),
            pl.BlockSpec((1, DH), lambda i: (0, 0)),
            pl.BlockSpec((1, DH), lambda i: (0, 0)),
            pl.BlockSpec((DH, 2 * DH), lambda i: (0, 0)),
            pl.BlockSpec((1, 2 * DH), lambda i: (0, 0)),
            pl.BlockSpec((2 * DH, DH), lambda i: (0, 0)),
            pl.BlockSpec((1, DH), lambda i: (0, 0)),
            pl.BlockSpec((1, DH), lambda i: (0, 0)),
            pl.BlockSpec((1, DH), lambda i: (0, 0)),
        ],
        out_specs=pl.BlockSpec((RB, DH), lambda i: (i, 0)),
        out_shape=jax.ShapeDtypeStruct((NT, DH), jnp.float32),
    )(h, nd, xl, xr, st, we, att, bias, gn, bn, w1, b1, w2, b2, gf, bf)


def _r2(v):
    return v.reshape(1, -1)


def kernel(x, edge_index, edge_attr, params):
    p = params
    src = edge_index[0].astype(jnp.int32)
    dst = edge_index[1].astype(jnp.int32)

    h = _k1(x.reshape(NT, IN_F), p['W_in'], _r2(p['b_in']), _r2(p['g_in']),
            _r2(p['be_in']))
    stats = _loopstats(dst, edge_attr)[:N]

    for k, (H, C, concat) in enumerate(LCFG):
        xl, xr = _k2(h, p['Wl%d' % k], _r2(p['bl%d' % k]),
                     p['Wr%d' % k], _r2(p['br%d' % k]))
        ep = _k3(edge_attr, p['We%d' % k])
        att = p['att%d' % k].reshape(-1)
        acc = _edge_kernel(C // 16, xl, xr, ep, att, src, dst)
        nd = acc[:, :, :CH, :].reshape(NT, ACC_W)
        h = _k4(H, h, nd, xl, xr, stats, p['We%d' % k], _r2(att),
                _r2(p['bias%d' % k]), _r2(p['g_n%d' % k]),
                _r2(p['b_n%d' % k]), p['W1_%d' % k], _r2(p['b1_%d' % k]),
                p['W2_%d' % k], _r2(p['b2_%d' % k]), _r2(p['g_f%d' % k]),
                _r2(p['b_f%d' % k]))
    return h.reshape(N, T, DH)
